# baseline (speedup 1.0000x reference)
import functools

import jax
import jax.numpy as jnp
from jax.experimental import pallas as pl

_B = 16
_HW = 1024
_K = 1024
_D = 64
_BN = 1024


def _vq_block(x_ref, w_ref, out_ref, sse_ref):
    b = pl.program_id(0)
    j = pl.program_id(1)
    xt = x_ref[0].reshape(_D, _BN)
    xb = xt.T
    w = w_ref[...]
    m2 = jax.lax.dot_general(
        xb * (-2.0), w, (((1,), (1,)), ((), ())),
        preferred_element_type=jnp.float32)
    fl = jnp.sum(xb * xb, axis=1, keepdims=True)
    w2 = jnp.sum(w * w, axis=1)
    d = (fl + w2) + m2
    dmin = jnp.min(d, axis=1, keepdims=True)
    kiota = jax.lax.broadcasted_iota(jnp.int32, (_BN, _K), 1)
    idx = jnp.min(jnp.where(d == dmin, kiota, _K), axis=1)
    onehot = (kiota == idx[:, None]).astype(jnp.bfloat16)
    q_t = jax.lax.dot_general(
        w.astype(jnp.bfloat16), onehot, (((0,), (1,)), ((), ())),
        preferred_element_type=jnp.float32)
    out_ref[...] = q_t[None]

    part = jnp.sum(dmin).reshape(1, 1)

    @pl.when((b == 0) & (j == 0))
    def _init():
        sse_ref[...] = jnp.zeros((1, 1), jnp.float32)

    sse_ref[...] += part


@functools.partial(jax.jit, static_argnames=())
def _vq_pallas(xv, weight):
    out, sse = pl.pallas_call(
        _vq_block,
        grid=(_B, _HW // _BN),
        in_specs=[
            pl.BlockSpec((1, _D, 32, 32), lambda b, j: (b, 0, 0, 0)),
            pl.BlockSpec((_K, _D), lambda b, j: (0, 0)),
        ],
        out_specs=[
            pl.BlockSpec((1, _D, _BN), lambda b, j: (b, 0, j)),
            pl.BlockSpec((1, 1), lambda b, j: (0, 0)),
        ],
        out_shape=[
            jax.ShapeDtypeStruct((_B, _D, _HW), jnp.float32),
            jax.ShapeDtypeStruct((1, 1), jnp.float32),
        ],
    )(xv, weight)
    return out, sse


def kernel(inputs, weight):
    out, sse = _vq_pallas(inputs, weight)
    mse = sse[0, 0] / (_B * _HW * _D)
    loss = mse + 0.25 * mse
    return (out.reshape(inputs.shape), loss)


# speedup vs baseline: 1.2364x; 1.2364x over previous
import functools

import jax
import jax.numpy as jnp
from jax.experimental import pallas as pl

_B = 16
_HW = 1024
_K = 1024
_D = 64
_BN = 1024


def _vq_block(x_ref, w_ref, out_ref, sse_ref):
    b = pl.program_id(0)
    j = pl.program_id(1)
    xt = x_ref[0]
    xb = xt.T
    w = w_ref[...]
    m2 = jax.lax.dot_general(
        xb * (-2.0), w, (((1,), (1,)), ((), ())),
        preferred_element_type=jnp.float32)
    fl = jnp.sum(xb * xb, axis=1, keepdims=True)
    w2 = jnp.sum(w * w, axis=1)
    d = (fl + w2) + m2
    dmin = jnp.min(d, axis=1, keepdims=True)
    kiota = jax.lax.broadcasted_iota(jnp.int32, (_BN, _K), 1)
    idx = jnp.min(jnp.where(d == dmin, kiota, _K), axis=1)
    onehot = (kiota == idx[:, None]).astype(jnp.bfloat16)
    q_t = jax.lax.dot_general(
        w.astype(jnp.bfloat16), onehot, (((0,), (1,)), ((), ())),
        preferred_element_type=jnp.float32)
    out_ref[...] = q_t[None]

    part = jnp.sum(dmin).reshape(1, 1)

    @pl.when((b == 0) & (j == 0))
    def _init():
        sse_ref[...] = jnp.zeros((1, 1), jnp.float32)

    sse_ref[...] += part


@functools.partial(jax.jit, static_argnames=())
def _vq_pallas(xv, weight):
    out, sse = pl.pallas_call(
        _vq_block,
        grid=(_B, _HW // _BN),
        in_specs=[
            pl.BlockSpec((1, _D, _BN), lambda b, j: (b, 0, j)),
            pl.BlockSpec((_K, _D), lambda b, j: (0, 0)),
        ],
        out_specs=[
            pl.BlockSpec((1, _D, _BN), lambda b, j: (b, 0, j)),
            pl.BlockSpec((1, 1), lambda b, j: (0, 0)),
        ],
        out_shape=[
            jax.ShapeDtypeStruct((_B, _D, _HW), jnp.float32),
            jax.ShapeDtypeStruct((1, 1), jnp.float32),
        ],
    )(xv, weight)
    return out, sse


def kernel(inputs, weight):
    xv = inputs.reshape(_B, _D, _HW)
    out, sse = _vq_pallas(xv, weight)
    mse = sse[0, 0] / (_B * _HW * _D)
    loss = mse + 0.25 * mse
    return (out.reshape(inputs.shape), loss)


# speedup vs baseline: 1.2748x; 1.0310x over previous
import functools

import jax
import jax.numpy as jnp
from jax.experimental import pallas as pl
from jax.experimental.pallas import tpu as pltpu

_B = 16
_HW = 1024
_K = 1024
_D = 64
_BN = 1024


def _vq_block(x_ref, w_ref, out_ref, loss_ref, acc_ref):
    b = pl.program_id(0)
    j = pl.program_id(1)
    xt = x_ref[0]
    xb = xt.T
    w = w_ref[...]
    m2 = jax.lax.dot_general(
        xb * (-2.0), w, (((1,), (1,)), ((), ())),
        preferred_element_type=jnp.float32)
    fl = jnp.sum(xb * xb, axis=1, keepdims=True)
    w2 = jnp.sum(w * w, axis=1)
    d = (fl + w2) + m2
    dmin = jnp.min(d, axis=1, keepdims=True)
    kiota = jax.lax.broadcasted_iota(jnp.int32, (_BN, _K), 1)
    idx = jnp.min(jnp.where(d == dmin, kiota, _K), axis=1)
    onehot = (kiota == idx[:, None]).astype(jnp.bfloat16)
    q_t = jax.lax.dot_general(
        w.astype(jnp.bfloat16), onehot, (((0,), (1,)), ((), ())),
        preferred_element_type=jnp.float32)
    out_ref[...] = q_t[None]

    part = jnp.sum(dmin).reshape(1, 1)

    @pl.when((b == 0) & (j == 0))
    def _init():
        acc_ref[...] = jnp.zeros((1, 1), jnp.float32)

    acc_ref[...] += part

    @pl.when((b == _B - 1) & (j == _HW // _BN - 1))
    def _fini():
        mse = acc_ref[...] / (_B * _HW * _D)
        loss_ref[...] = mse + 0.25 * mse


@functools.partial(jax.jit, static_argnames=())
def _vq_pallas(xv, weight):
    out, loss = pl.pallas_call(
        _vq_block,
        grid=(_B, _HW // _BN),
        in_specs=[
            pl.BlockSpec((1, _D, _BN), lambda b, j: (b, 0, j)),
            pl.BlockSpec((_K, _D), lambda b, j: (0, 0)),
        ],
        out_specs=[
            pl.BlockSpec((1, _D, _BN), lambda b, j: (b, 0, j)),
            pl.BlockSpec((1, 1), lambda b, j: (0, 0)),
        ],
        out_shape=[
            jax.ShapeDtypeStruct((_B, _D, _HW), jnp.float32),
            jax.ShapeDtypeStruct((1, 1), jnp.float32),
        ],
        scratch_shapes=[pltpu.VMEM((1, 1), jnp.float32)],
    )(xv, weight)
    return out, loss


def kernel(inputs, weight):
    xv = inputs.reshape(_B, _D, _HW)
    out, loss = _vq_pallas(xv, weight)
    return (out.reshape(inputs.shape), loss[0, 0])
